# baseline (speedup 1.0000x reference)
import functools

import jax
import jax.numpy as jnp
from jax import lax
from jax.experimental import pallas as pl
from jax.experimental.pallas import tpu as pltpu
from jax.experimental.pallas import tpu_sc as plsc

N_SRC = 50000
N_MID = 20000
N_DST = 4096
FANOUT = 16
D = 128
H = 128

_NC = 2
_NS = 16
_NW = _NC * _NS
_CHUNK = 128


def _sc_gather(table, idx_flat):
    B = idx_flat.shape[0]
    assert B % (_NW * _CHUNK) == 0, B
    n_chunks = B // (_NW * _CHUNK)
    d = table.shape[1]
    idx3d = idx_flat.reshape(n_chunks, _NW, _CHUNK)

    mesh = plsc.VectorSubcoreMesh(core_axis_name="c", subcore_axis_name="s")

    _NBUF = 4
    assert n_chunks % _NBUF == 0, n_chunks

    def body(table_hbm, idx_hbm, out_hbm, idx_v, *rest):
        bufs = rest[:_NBUF]
        gsems = rest[_NBUF:2 * _NBUF]
        wsems = rest[2 * _NBUF:3 * _NBUF]
        wid = lax.axis_index("s") * _NC + lax.axis_index("c")
        pltpu.sync_copy(idx_hbm.at[:, wid], idx_v)

        def fire(j, p):
            pltpu.make_async_copy(table_hbm.at[idx_v.at[j]], bufs[p],
                                  gsems[p]).start()

        def out_slice(j):
            return out_hbm.at[pl.ds((j * _NW + wid) * _CHUNK, _CHUNK)]

        for p in range(_NBUF):
            fire(p, p)

        def step(i, carry):
            for p in range(_NBUF):
                j = _NBUF * i + p
                pltpu.make_async_copy(table_hbm.at[idx_v.at[0]], bufs[p],
                                      gsems[p]).wait()
                wb = pltpu.make_async_copy(bufs[p], out_slice(j), wsems[p])
                wb.start()
                wb.wait()

                @pl.when(j + _NBUF < n_chunks)
                def _():
                    fire(j + _NBUF, p)
            return carry

        lax.fori_loop(0, n_chunks // _NBUF, step, 0)

    gather = pl.kernel(
        body,
        out_type=jax.ShapeDtypeStruct((B, d), table.dtype),
        mesh=mesh,
        scratch_types=(
            [pltpu.VMEM((n_chunks, _CHUNK), jnp.int32)]
            + [pltpu.VMEM((_CHUNK, d), table.dtype) for _ in range(_NBUF)]
            + [pltpu.SemaphoreType.DMA for _ in range(2 * _NBUF)]
        ),
    )
    return gather(table, idx3d)


def _gelu(x):
    return 0.5 * x * (1.0 + lax.erf(x * (2.0 ** -0.5)))


def _lstm_block(g_ref, hdst_ref, wcat_ref, bg_ref, wdn_ref, bout_ref):
    bn = g_ref.shape[1]
    h = jnp.zeros((bn, H), jnp.float32)
    c = jnp.zeros((bn, H), jnp.float32)
    for t in range(FANOUT):
        xh = jnp.concatenate([g_ref[t].astype(jnp.bfloat16),
                              h.astype(jnp.bfloat16)], axis=1)
        gates = jnp.dot(xh, wcat_ref[...], preferred_element_type=jnp.float32)
        gates = gates + bg_ref[...]
        i = jax.nn.sigmoid(gates[:, 0:H])
        f = jax.nn.sigmoid(gates[:, H:2 * H])
        g = jnp.tanh(gates[:, 2 * H:3 * H])
        o = jax.nn.sigmoid(gates[:, 3 * H:4 * H])
        c = f * c + i * g
        h = o * jnp.tanh(c)
    dh = jnp.concatenate([hdst_ref[...], h], axis=1)
    out = jnp.dot(dh, wdn_ref[...], preferred_element_type=jnp.float32)
    return _gelu(out + bout_ref[...])


def _layernorm(x, g, b):
    m = jnp.mean(x, axis=-1, keepdims=True)
    xc = x - m
    v = jnp.mean(xc * xc, axis=-1, keepdims=True)
    return xc * lax.rsqrt(v + 1e-5) * g + b


def _conv_body(g_ref, hdst_ref, wcat_ref, bg_ref, wdn_ref, bout_ref,
               lng_ref, lnb_ref, out_ref):
    out = _lstm_block(g_ref, hdst_ref, wcat_ref, bg_ref, wdn_ref, bout_ref)
    y = _layernorm(out, lng_ref[...], lnb_ref[...])
    out_ref[...] = y.astype(out_ref.dtype)


def _conv_head_body(g_ref, hdst_ref, wcat_ref, bg_ref, wdn_ref, bout_ref,
                    lng_ref, lnb_ref, sg_ref, sb_ref, w1t_ref, b1_ref,
                    w2_ref, out_ref):
    out = _lstm_block(g_ref, hdst_ref, wcat_ref, bg_ref, wdn_ref, bout_ref)
    y = _layernorm(out, lng_ref[...], lnb_ref[...])
    hb = y * sg_ref[...] + sb_ref[...]
    h1 = _gelu(jnp.dot(hb, w1t_ref[...], preferred_element_type=jnp.float32)
               + b1_ref[...])
    out_ref[...] = jnp.sum(h1 * w2_ref[...], axis=1, keepdims=True)


def _row(v):
    return v.reshape(1, -1)


def _conv_weights(p):
    wcat = jnp.concatenate([p["W_ih"].T, p["W_hh"].T], axis=0)
    bg = _row(p["b_ih"] + p["b_hh"])
    wdn = jnp.concatenate([p["W_self"].T, p["W_neigh"].T], axis=0)
    bout = _row(p["b_self"] + p["b_neigh"])
    return wcat.astype(jnp.bfloat16), bg, wdn, bout


def _full(shape):
    return pl.BlockSpec(shape, lambda i: (0,) * len(shape))


def _conv_call(g, hdst, wts, lng, lnb, bn_rows, head=None, out_dtype=jnp.float32):
    t, n, d = g.shape
    grid = n // bn_rows
    in_specs = [
        pl.BlockSpec((t, bn_rows, d), lambda i: (0, i, 0)),
        pl.BlockSpec((bn_rows, d), lambda i: (i, 0)),
        _full((2 * D, 4 * H)),
        _full((1, 4 * H)),
        _full((2 * H, H)),
        _full((1, H)),
        _full((1, H)),
        _full((1, H)),
    ]
    args = [g, hdst, *wts, _row(lng), _row(lnb)]
    if head is None:
        body = _conv_body
        out_shape = jax.ShapeDtypeStruct((n, H), out_dtype)
        out_spec = pl.BlockSpec((bn_rows, H), lambda i: (i, 0))
    else:
        sg, sb, w1t, b1, w2 = head
        body = _conv_head_body
        in_specs += [_full((1, H)), _full((1, H)), _full((H, H)),
                     _full((1, H)), _full((1, H))]
        args += [_row(sg), _row(sb), w1t, _row(b1), _row(w2)]
        out_shape = jax.ShapeDtypeStruct((n, 1), jnp.float32)
        out_spec = pl.BlockSpec((bn_rows, 1), lambda i: (i, 0))
    return pl.pallas_call(
        body,
        grid=(grid,),
        in_specs=in_specs,
        out_specs=out_spec,
        out_shape=out_shape,
        compiler_params=pltpu.CompilerParams(
            dimension_semantics=("arbitrary",),
        ),
    )(*args)


def _tmajor(idx):
    return idx.astype(jnp.int32).T.reshape(-1)


def kernel(x_user, x_message, params, neigh_u2m_0, neigh_m2u_0,
           neigh_u2m_1, neigh_m2u_1):
    p = params
    bn1 = 512
    n_mid_pad = 20480

    idx1 = jnp.pad(neigh_m2u_0.astype(jnp.int32), ((0, n_mid_pad - N_MID), (0, 0)))
    g1 = _sc_gather(x_message, idx1.T.reshape(-1)).reshape(FANOUT, n_mid_pad, D)
    g2 = _sc_gather(x_user, _tmajor(neigh_u2m_0[:N_DST])).reshape(FANOUT, N_DST, D)

    w_m2u = _conv_weights(p["conv1"]["m2u"])
    w_u2m = _conv_weights(p["conv1"]["u2m"])
    lng, lnb = p["ln_g"], p["ln_b"]
    hdst1 = jnp.pad(x_user[:N_MID], ((0, n_mid_pad - N_MID), (0, 0)))
    usr1 = _conv_call(g1, hdst1, w_m2u, lng, lnb, bn1)
    msg1 = _conv_call(g2, x_message[:N_DST], w_u2m, lng, lnb, bn1)

    g3 = _sc_gather(usr1, _tmajor(neigh_u2m_1)).reshape(FANOUT, N_DST, D)
    w2l = _conv_weights(p["conv2"]["u2m"])
    bn_scale = 1.0 / jnp.sqrt(jnp.float32(1.0 + 1e-5))
    head = (p["bn_g"] * bn_scale, p["bn_b"], p["W1"].T, p["b1"], p["W2"][0])
    logits = _conv_call(g3, msg1, w2l, lng, lnb, bn1, head=head)
    return logits + p["b2"]


# speedup vs baseline: 1.7575x; 1.7575x over previous
import functools

import jax
import jax.numpy as jnp
from jax import lax
from jax.experimental import pallas as pl
from jax.experimental.pallas import tpu as pltpu
from jax.experimental.pallas import tpu_sc as plsc

N_SRC = 50000
N_MID = 20000
N_DST = 4096
FANOUT = 16
D = 128
H = 128

_NC = 2
_NS = 16
_NW = _NC * _NS
_CHUNK = 128


def _sc_gather(table, idx_flat):
    B = idx_flat.shape[0]
    assert B % (_NW * _CHUNK) == 0, B
    n_chunks = B // (_NW * _CHUNK)
    d = table.shape[1]
    idx2d = idx_flat.reshape(B // _CHUNK, _CHUNK)

    mesh = plsc.VectorSubcoreMesh(core_axis_name="c", subcore_axis_name="s")

    _NBUF = 4
    assert n_chunks % _NBUF == 0, n_chunks

    def body(table_hbm, idx_hbm, out_hbm, idx_v, *rest):
        bufs = rest[:_NBUF]
        gsems = rest[_NBUF:2 * _NBUF]
        wsems = rest[2 * _NBUF:3 * _NBUF]
        wid = lax.axis_index("s") * _NC + lax.axis_index("c")
        pltpu.sync_copy(idx_hbm.at[pl.ds(wid * n_chunks, n_chunks)], idx_v)

        def fire(j, p):
            pltpu.make_async_copy(table_hbm.at[idx_v.at[j]], bufs[p],
                                  gsems[p]).start()

        def out_slice(j):
            return out_hbm.at[pl.ds((wid * n_chunks + j) * _CHUNK, _CHUNK)]

        for p in range(_NBUF):
            fire(p, p)

        def step(i, carry):
            for p in range(_NBUF):
                j = _NBUF * i + p
                pltpu.make_async_copy(table_hbm.at[idx_v.at[0]], bufs[p],
                                      gsems[p]).wait()
                wb = pltpu.make_async_copy(bufs[p], out_slice(j), wsems[p])
                wb.start()
                wb.wait()

                @pl.when(j + _NBUF < n_chunks)
                def _():
                    fire(j + _NBUF, p)
            return carry

        lax.fori_loop(0, n_chunks // _NBUF, step, 0)

    gather = pl.kernel(
        body,
        out_type=jax.ShapeDtypeStruct((B, d), table.dtype),
        mesh=mesh,
        scratch_types=(
            [pltpu.VMEM((n_chunks, _CHUNK), jnp.int32)]
            + [pltpu.VMEM((_CHUNK, d), table.dtype) for _ in range(_NBUF)]
            + [pltpu.SemaphoreType.DMA for _ in range(2 * _NBUF)]
        ),
    )
    return gather(table, idx2d)


def _gelu(x):
    return 0.5 * x * (1.0 + lax.erf(x * (2.0 ** -0.5)))


def _lstm_block(g_ref, hdst_ref, wcat_ref, bg_ref, wdn_ref, bout_ref):
    bn = g_ref.shape[1]
    h = jnp.zeros((bn, H), jnp.float32)
    c = jnp.zeros((bn, H), jnp.float32)
    for t in range(FANOUT):
        xh = jnp.concatenate([g_ref[t].astype(jnp.bfloat16),
                              h.astype(jnp.bfloat16)], axis=1)
        gates = jnp.dot(xh, wcat_ref[...], preferred_element_type=jnp.float32)
        gates = gates + bg_ref[...]
        i = jax.nn.sigmoid(gates[:, 0:H])
        f = jax.nn.sigmoid(gates[:, H:2 * H])
        g = jnp.tanh(gates[:, 2 * H:3 * H])
        o = jax.nn.sigmoid(gates[:, 3 * H:4 * H])
        c = f * c + i * g
        h = o * jnp.tanh(c)
    dh = jnp.concatenate([hdst_ref[...], h], axis=1)
    out = jnp.dot(dh, wdn_ref[...], preferred_element_type=jnp.float32)
    return _gelu(out + bout_ref[...])


def _layernorm(x, g, b):
    m = jnp.mean(x, axis=-1, keepdims=True)
    xc = x - m
    v = jnp.mean(xc * xc, axis=-1, keepdims=True)
    return xc * lax.rsqrt(v + 1e-5) * g + b


def _conv_body(g_ref, hdst_ref, wcat_ref, bg_ref, wdn_ref, bout_ref,
               lng_ref, lnb_ref, out_ref):
    out = _lstm_block(g_ref, hdst_ref, wcat_ref, bg_ref, wdn_ref, bout_ref)
    y = _layernorm(out, lng_ref[...], lnb_ref[...])
    out_ref[...] = y.astype(out_ref.dtype)


def _conv_head_body(g_ref, hdst_ref, wcat_ref, bg_ref, wdn_ref, bout_ref,
                    lng_ref, lnb_ref, sg_ref, sb_ref, w1t_ref, b1_ref,
                    w2_ref, out_ref):
    out = _lstm_block(g_ref, hdst_ref, wcat_ref, bg_ref, wdn_ref, bout_ref)
    y = _layernorm(out, lng_ref[...], lnb_ref[...])
    hb = y * sg_ref[...] + sb_ref[...]
    h1 = _gelu(jnp.dot(hb, w1t_ref[...], preferred_element_type=jnp.float32)
               + b1_ref[...])
    out_ref[...] = jnp.sum(h1 * w2_ref[...], axis=1, keepdims=True)


def _row(v):
    return v.reshape(1, -1)


def _conv_weights(p):
    wcat = jnp.concatenate([p["W_ih"].T, p["W_hh"].T], axis=0)
    bg = _row(p["b_ih"] + p["b_hh"])
    wdn = jnp.concatenate([p["W_self"].T, p["W_neigh"].T], axis=0)
    bout = _row(p["b_self"] + p["b_neigh"])
    return wcat.astype(jnp.bfloat16), bg, wdn, bout


def _full(shape):
    return pl.BlockSpec(shape, lambda i: (0,) * len(shape))


def _conv_call(g, hdst, wts, lng, lnb, bn_rows, head=None, out_dtype=jnp.float32):
    t, n, d = g.shape
    grid = n // bn_rows
    in_specs = [
        pl.BlockSpec((t, bn_rows, d), lambda i: (0, i, 0)),
        pl.BlockSpec((bn_rows, d), lambda i: (i, 0)),
        _full((2 * D, 4 * H)),
        _full((1, 4 * H)),
        _full((2 * H, H)),
        _full((1, H)),
        _full((1, H)),
        _full((1, H)),
    ]
    args = [g, hdst, *wts, _row(lng), _row(lnb)]
    if head is None:
        body = _conv_body
        out_shape = jax.ShapeDtypeStruct((n, H), out_dtype)
        out_spec = pl.BlockSpec((bn_rows, H), lambda i: (i, 0))
    else:
        sg, sb, w1t, b1, w2 = head
        body = _conv_head_body
        in_specs += [_full((1, H)), _full((1, H)), _full((H, H)),
                     _full((1, H)), _full((1, H))]
        args += [_row(sg), _row(sb), w1t, _row(b1), _row(w2)]
        out_shape = jax.ShapeDtypeStruct((n, 1), jnp.float32)
        out_spec = pl.BlockSpec((bn_rows, 1), lambda i: (i, 0))
    return pl.pallas_call(
        body,
        grid=(grid,),
        in_specs=in_specs,
        out_specs=out_spec,
        out_shape=out_shape,
        compiler_params=pltpu.CompilerParams(
            dimension_semantics=("arbitrary",),
        ),
    )(*args)


def _tmajor(idx):
    return idx.astype(jnp.int32).T.reshape(-1)


def kernel(x_user, x_message, params, neigh_u2m_0, neigh_m2u_0,
           neigh_u2m_1, neigh_m2u_1):
    p = params
    bn1 = 512
    n_mid_pad = 20480

    n_pad_rows = n_mid_pad - N_MID
    pad_idx = (jnp.arange(n_pad_rows * FANOUT, dtype=jnp.int32) * 65)[:, None]
    pad_idx = pad_idx.reshape(n_pad_rows, FANOUT) % N_SRC
    idx1 = jnp.concatenate([neigh_m2u_0.astype(jnp.int32), pad_idx], axis=0)
    g1 = _sc_gather(x_message, idx1.T.reshape(-1)).reshape(FANOUT, n_mid_pad, D)
    g2 = _sc_gather(x_user, _tmajor(neigh_u2m_0[:N_DST])).reshape(FANOUT, N_DST, D)

    w_m2u = _conv_weights(p["conv1"]["m2u"])
    w_u2m = _conv_weights(p["conv1"]["u2m"])
    lng, lnb = p["ln_g"], p["ln_b"]
    hdst1 = jnp.pad(x_user[:N_MID], ((0, n_mid_pad - N_MID), (0, 0)))
    usr1 = _conv_call(g1, hdst1, w_m2u, lng, lnb, bn1)
    msg1 = _conv_call(g2, x_message[:N_DST], w_u2m, lng, lnb, bn1)

    g3 = _sc_gather(usr1, _tmajor(neigh_u2m_1)).reshape(FANOUT, N_DST, D)
    w2l = _conv_weights(p["conv2"]["u2m"])
    bn_scale = 1.0 / jnp.sqrt(jnp.float32(1.0 + 1e-5))
    head = (p["bn_g"] * bn_scale, p["bn_b"], p["W1"].T, p["b1"], p["W2"][0])
    logits = _conv_call(g3, msg1, w2l, lng, lnb, bn1, head=head)
    return logits + p["b2"]


# speedup vs baseline: 2.0314x; 1.1558x over previous
import functools

import jax
import jax.numpy as jnp
from jax import lax
from jax.experimental import pallas as pl
from jax.experimental.pallas import tpu as pltpu
from jax.experimental.pallas import tpu_sc as plsc

N_SRC = 50000
N_MID = 20000
N_DST = 4096
FANOUT = 16
D = 128
H = 128

_NC = 2
_NS = 16
_NW = _NC * _NS
_CHUNK = 128


def _sc_gather(table, idx_flat):
    B = idx_flat.shape[0]
    assert B % (_NW * _CHUNK) == 0, B
    n_chunks = B // (_NW * _CHUNK)
    d = table.shape[1]
    idx2d = idx_flat.reshape(B // _CHUNK, _CHUNK)

    mesh = plsc.VectorSubcoreMesh(core_axis_name="c", subcore_axis_name="s")

    _NBUF = 4
    assert n_chunks % _NBUF == 0, n_chunks

    def body(table_hbm, idx_hbm, out_hbm, idx_v, *rest):
        bufs = rest[:_NBUF]
        gsems = rest[_NBUF:2 * _NBUF]
        wsems = rest[2 * _NBUF:3 * _NBUF]
        wid = lax.axis_index("s") * _NC + lax.axis_index("c")
        pltpu.sync_copy(idx_hbm.at[pl.ds(wid * n_chunks, n_chunks)], idx_v)

        def fire(j, p):
            pltpu.make_async_copy(table_hbm.at[idx_v.at[j]], bufs[p],
                                  gsems[p]).start()

        def out_slice(j):
            return out_hbm.at[pl.ds((wid * n_chunks + j) * _CHUNK, _CHUNK)]

        for p in range(_NBUF):
            fire(p, p)

        def step(i, carry):
            for p in range(_NBUF):
                j = _NBUF * i + p
                pltpu.make_async_copy(table_hbm.at[idx_v.at[0]], bufs[p],
                                      gsems[p]).wait()
                wb = pltpu.make_async_copy(bufs[p], out_slice(j), wsems[p])
                wb.start()
                wb.wait()

                @pl.when(j + _NBUF < n_chunks)
                def _():
                    fire(j + _NBUF, p)
            return carry

        lax.fori_loop(0, n_chunks // _NBUF, step, 0)

    gather = pl.kernel(
        body,
        out_type=jax.ShapeDtypeStruct((B, d), table.dtype),
        mesh=mesh,
        scratch_types=(
            [pltpu.VMEM((n_chunks, _CHUNK), jnp.int32)]
            + [pltpu.VMEM((_CHUNK, d), table.dtype) for _ in range(_NBUF)]
            + [pltpu.SemaphoreType.DMA for _ in range(2 * _NBUF)]
        ),
    )
    return gather(table, idx2d)


def _gelu(x):
    return 0.5 * x * (1.0 + lax.erf(x * (2.0 ** -0.5)))


def _sigmoid(x):
    return 0.5 * (1.0 + jnp.tanh(0.5 * x))


def _lstm_block(g_ref, hdst_ref, wcat_ref, bg_ref, wdn_ref, bout_ref):
    bn = g_ref.shape[1]
    h = jnp.zeros((bn, H), jnp.float32)
    c = jnp.zeros((bn, H), jnp.float32)
    for t in range(FANOUT):
        xh = jnp.concatenate([g_ref[t].astype(jnp.bfloat16),
                              h.astype(jnp.bfloat16)], axis=1)
        gates = jnp.dot(xh, wcat_ref[...], preferred_element_type=jnp.float32)
        gates = gates + bg_ref[...]
        i = _sigmoid(gates[:, 0:H])
        f = _sigmoid(gates[:, H:2 * H])
        g = jnp.tanh(gates[:, 2 * H:3 * H])
        o = _sigmoid(gates[:, 3 * H:4 * H])
        c = f * c + i * g
        h = o * jnp.tanh(c)
    dh = jnp.concatenate([hdst_ref[...], h], axis=1)
    out = jnp.dot(dh, wdn_ref[...], preferred_element_type=jnp.float32)
    return _gelu(out + bout_ref[...])


def _layernorm(x, g, b):
    m = jnp.mean(x, axis=-1, keepdims=True)
    xc = x - m
    v = jnp.mean(xc * xc, axis=-1, keepdims=True)
    return xc * lax.rsqrt(v + 1e-5) * g + b


def _conv_body(g_ref, hdst_ref, wcat_ref, bg_ref, wdn_ref, bout_ref,
               lng_ref, lnb_ref, out_ref):
    out = _lstm_block(g_ref, hdst_ref, wcat_ref, bg_ref, wdn_ref, bout_ref)
    y = _layernorm(out, lng_ref[...], lnb_ref[...])
    out_ref[...] = y.astype(out_ref.dtype)


def _conv_head_body(g_ref, hdst_ref, wcat_ref, bg_ref, wdn_ref, bout_ref,
                    lng_ref, lnb_ref, sg_ref, sb_ref, w1t_ref, b1_ref,
                    w2_ref, out_ref):
    out = _lstm_block(g_ref, hdst_ref, wcat_ref, bg_ref, wdn_ref, bout_ref)
    y = _layernorm(out, lng_ref[...], lnb_ref[...])
    hb = y * sg_ref[...] + sb_ref[...]
    h1 = _gelu(jnp.dot(hb, w1t_ref[...], preferred_element_type=jnp.float32)
               + b1_ref[...])
    out_ref[...] = jnp.sum(h1 * w2_ref[...], axis=1, keepdims=True)


def _row(v):
    return v.reshape(1, -1)


def _conv_weights(p):
    wcat = jnp.concatenate([p["W_ih"].T, p["W_hh"].T], axis=0)
    bg = _row(p["b_ih"] + p["b_hh"])
    wdn = jnp.concatenate([p["W_self"].T, p["W_neigh"].T], axis=0)
    bout = _row(p["b_self"] + p["b_neigh"])
    return wcat.astype(jnp.bfloat16), bg, wdn, bout


def _full(shape):
    return pl.BlockSpec(shape, lambda i: (0,) * len(shape))


def _conv_call(g, hdst, wts, lng, lnb, bn_rows, head=None, out_dtype=jnp.float32):
    t, n, d = g.shape
    grid = n // bn_rows
    in_specs = [
        pl.BlockSpec((t, bn_rows, d), lambda i: (0, i, 0)),
        pl.BlockSpec((bn_rows, d), lambda i: (i, 0)),
        _full((2 * D, 4 * H)),
        _full((1, 4 * H)),
        _full((2 * H, H)),
        _full((1, H)),
        _full((1, H)),
        _full((1, H)),
    ]
    args = [g, hdst, *wts, _row(lng), _row(lnb)]
    if head is None:
        body = _conv_body
        out_shape = jax.ShapeDtypeStruct((n, H), out_dtype)
        out_spec = pl.BlockSpec((bn_rows, H), lambda i: (i, 0))
    else:
        sg, sb, w1t, b1, w2 = head
        body = _conv_head_body
        in_specs += [_full((1, H)), _full((1, H)), _full((H, H)),
                     _full((1, H)), _full((1, H))]
        args += [_row(sg), _row(sb), w1t, _row(b1), _row(w2)]
        out_shape = jax.ShapeDtypeStruct((n, 1), jnp.float32)
        out_spec = pl.BlockSpec((bn_rows, 1), lambda i: (i, 0))
    return pl.pallas_call(
        body,
        grid=(grid,),
        in_specs=in_specs,
        out_specs=out_spec,
        out_shape=out_shape,
        compiler_params=pltpu.CompilerParams(
            dimension_semantics=("arbitrary",),
        ),
    )(*args)


def _tmajor(idx):
    return idx.astype(jnp.int32).T.reshape(-1)


def kernel(x_user, x_message, params, neigh_u2m_0, neigh_m2u_0,
           neigh_u2m_1, neigh_m2u_1):
    p = params
    bn1 = 512
    n_mid_pad = 20480

    n_pad_rows = n_mid_pad - N_MID
    pad_idx = (jnp.arange(n_pad_rows * FANOUT, dtype=jnp.int32) * 65)[:, None]
    pad_idx = pad_idx.reshape(n_pad_rows, FANOUT) % N_SRC
    idx1 = jnp.concatenate([neigh_m2u_0.astype(jnp.int32), pad_idx], axis=0)
    g1 = _sc_gather(x_message, idx1.T.reshape(-1)).reshape(FANOUT, n_mid_pad, D)
    g2 = _sc_gather(x_user, _tmajor(neigh_u2m_0[:N_DST])).reshape(FANOUT, N_DST, D)

    w_m2u = _conv_weights(p["conv1"]["m2u"])
    w_u2m = _conv_weights(p["conv1"]["u2m"])
    lng, lnb = p["ln_g"], p["ln_b"]
    hdst1 = jnp.pad(x_user[:N_MID], ((0, n_mid_pad - N_MID), (0, 0)))
    usr1 = _conv_call(g1, hdst1, w_m2u, lng, lnb, bn1)
    msg1 = _conv_call(g2, x_message[:N_DST], w_u2m, lng, lnb, bn1)

    g3 = _sc_gather(usr1, _tmajor(neigh_u2m_1)).reshape(FANOUT, N_DST, D)
    w2l = _conv_weights(p["conv2"]["u2m"])
    bn_scale = 1.0 / jnp.sqrt(jnp.float32(1.0 + 1e-5))
    head = (p["bn_g"] * bn_scale, p["bn_b"], p["W1"].T, p["b1"], p["W2"][0])
    logits = _conv_call(g3, msg1, w2l, lng, lnb, bn1, head=head)
    return logits + p["b2"]


# speedup vs baseline: 2.1456x; 1.0562x over previous
import functools

import jax
import jax.numpy as jnp
from jax import lax
from jax.experimental import pallas as pl
from jax.experimental.pallas import tpu as pltpu
from jax.experimental.pallas import tpu_sc as plsc

N_SRC = 50000
N_MID = 20000
N_DST = 4096
FANOUT = 16
D = 128
H = 128

_NC = 2
_NS = 16
_NW = _NC * _NS
_CHUNK = 128


def _sc_gather(table, idx_flat):
    B = idx_flat.shape[0]
    assert B % (_NW * _CHUNK) == 0, B
    n_chunks = B // (_NW * _CHUNK)
    d = table.shape[1]
    idx2d = idx_flat.reshape(B // _CHUNK, _CHUNK)

    mesh = plsc.VectorSubcoreMesh(core_axis_name="c", subcore_axis_name="s")

    _NBUF = 4
    assert n_chunks % _NBUF == 0, n_chunks

    def body(table_hbm, idx_hbm, out_hbm, idx_v, *rest):
        bufs = rest[:_NBUF]
        gsems = rest[_NBUF:2 * _NBUF]
        wsems = rest[2 * _NBUF:3 * _NBUF]
        wid = lax.axis_index("s") * _NC + lax.axis_index("c")
        pltpu.sync_copy(idx_hbm.at[pl.ds(wid * n_chunks, n_chunks)], idx_v)

        def fire(j, p):
            pltpu.make_async_copy(table_hbm.at[idx_v.at[j]], bufs[p],
                                  gsems[p]).start()

        def out_slice(j):
            return out_hbm.at[pl.ds((wid * n_chunks + j) * _CHUNK, _CHUNK)]

        for p in range(_NBUF):
            fire(p, p)

        def step(i, carry):
            for p in range(_NBUF):
                j = _NBUF * i + p
                pltpu.make_async_copy(table_hbm.at[idx_v.at[0]], bufs[p],
                                      gsems[p]).wait()
                wb = pltpu.make_async_copy(bufs[p], out_slice(j), wsems[p])
                wb.start()
                wb.wait()

                @pl.when(j + _NBUF < n_chunks)
                def _():
                    fire(j + _NBUF, p)
            return carry

        lax.fori_loop(0, n_chunks // _NBUF, step, 0)

    gather = pl.kernel(
        body,
        out_type=jax.ShapeDtypeStruct((B, d), table.dtype),
        mesh=mesh,
        scratch_types=(
            [pltpu.VMEM((n_chunks, _CHUNK), jnp.int32)]
            + [pltpu.VMEM((_CHUNK, d), table.dtype) for _ in range(_NBUF)]
            + [pltpu.SemaphoreType.DMA for _ in range(2 * _NBUF)]
        ),
    )
    return gather(table, idx2d)


def _gelu(x):
    return 0.5 * x * (1.0 + lax.erf(x * (2.0 ** -0.5)))


def _sigmoid(x):
    return 0.5 * (1.0 + jnp.tanh(0.5 * x))


def _lstm_block(g_ref, hdst_ref, wcat_ref, bg_ref, wdn_ref, bout_ref):
    bn = g_ref.shape[1]
    h = jnp.zeros((bn, H), jnp.float32)
    c = jnp.zeros((bn, H), jnp.float32)
    for t in range(FANOUT):
        xh = jnp.concatenate([g_ref[t].astype(jnp.bfloat16),
                              h.astype(jnp.bfloat16)], axis=1)
        gates = jnp.dot(xh, wcat_ref[...], preferred_element_type=jnp.float32)
        gates = gates + bg_ref[...]
        i = _sigmoid(gates[:, 0:H])
        f = _sigmoid(gates[:, H:2 * H])
        g = jnp.tanh(gates[:, 2 * H:3 * H])
        o = _sigmoid(gates[:, 3 * H:4 * H])
        c = f * c + i * g
        h = o * jnp.tanh(c)
    dh = jnp.concatenate([hdst_ref[...], h], axis=1)
    out = jnp.dot(dh, wdn_ref[...], preferred_element_type=jnp.float32)
    return _gelu(out + bout_ref[...])


def _layernorm(x, g, b):
    m = jnp.mean(x, axis=-1, keepdims=True)
    xc = x - m
    v = jnp.mean(xc * xc, axis=-1, keepdims=True)
    return xc * lax.rsqrt(v + 1e-5) * g + b


def _conv_body(g_ref, hdst_ref, wcat_ref, bg_ref, wdn_ref, bout_ref,
               lng_ref, lnb_ref, out_ref):
    out = _lstm_block(g_ref, hdst_ref, wcat_ref, bg_ref, wdn_ref, bout_ref)
    y = _layernorm(out, lng_ref[...], lnb_ref[...])
    out_ref[...] = y.astype(out_ref.dtype)


def _conv_head_body(g_ref, hdst_ref, wcat_ref, bg_ref, wdn_ref, bout_ref,
                    lng_ref, lnb_ref, sg_ref, sb_ref, w1t_ref, b1_ref,
                    w2_ref, out_ref):
    out = _lstm_block(g_ref, hdst_ref, wcat_ref, bg_ref, wdn_ref, bout_ref)
    y = _layernorm(out, lng_ref[...], lnb_ref[...])
    hb = y * sg_ref[...] + sb_ref[...]
    h1 = _gelu(jnp.dot(hb, w1t_ref[...], preferred_element_type=jnp.float32)
               + b1_ref[...])
    out_ref[...] = jnp.sum(h1 * w2_ref[...], axis=1, keepdims=True)


def _row(v):
    return v.reshape(1, -1)


def _conv_weights(p):
    wcat = jnp.concatenate([p["W_ih"].T, p["W_hh"].T], axis=0)
    bg = _row(p["b_ih"] + p["b_hh"])
    wdn = jnp.concatenate([p["W_self"].T, p["W_neigh"].T], axis=0)
    bout = _row(p["b_self"] + p["b_neigh"])
    return wcat.astype(jnp.bfloat16), bg, wdn, bout


def _full(shape):
    return pl.BlockSpec(shape, lambda i: (0,) * len(shape))


def _conv_call(g, hdst, wts, lng, lnb, bn_rows, head=None, out_dtype=jnp.float32):
    t, n, d = g.shape
    grid = n // bn_rows
    in_specs = [
        pl.BlockSpec((t, bn_rows, d), lambda i: (0, i, 0)),
        pl.BlockSpec((bn_rows, d), lambda i: (i, 0)),
        _full((2 * D, 4 * H)),
        _full((1, 4 * H)),
        _full((2 * H, H)),
        _full((1, H)),
        _full((1, H)),
        _full((1, H)),
    ]
    args = [g, hdst, *wts, _row(lng), _row(lnb)]
    if head is None:
        body = _conv_body
        out_shape = jax.ShapeDtypeStruct((n, H), out_dtype)
        out_spec = pl.BlockSpec((bn_rows, H), lambda i: (i, 0))
    else:
        sg, sb, w1t, b1, w2 = head
        body = _conv_head_body
        in_specs += [_full((1, H)), _full((1, H)), _full((H, H)),
                     _full((1, H)), _full((1, H))]
        args += [_row(sg), _row(sb), w1t, _row(b1), _row(w2)]
        out_shape = jax.ShapeDtypeStruct((n, 1), jnp.float32)
        out_spec = pl.BlockSpec((bn_rows, 1), lambda i: (i, 0))
    return pl.pallas_call(
        body,
        grid=(grid,),
        in_specs=in_specs,
        out_specs=out_spec,
        out_shape=out_shape,
        compiler_params=pltpu.CompilerParams(
            dimension_semantics=("arbitrary",),
        ),
    )(*args)


def _tmajor(idx):
    return idx.astype(jnp.int32).T.reshape(-1)


def kernel(x_user, x_message, params, neigh_u2m_0, neigh_m2u_0,
           neigh_u2m_1, neigh_m2u_1):
    p = params
    bn1 = 512
    n_mid_pad = 20480

    n_pad_rows = n_mid_pad - N_MID
    pad_idx = (jnp.arange(n_pad_rows * FANOUT, dtype=jnp.int32) * 65)[:, None]
    pad_idx = pad_idx.reshape(n_pad_rows, FANOUT) % N_SRC
    idx1 = jnp.concatenate([neigh_m2u_0.astype(jnp.int32), pad_idx], axis=0)
    g2 = _sc_gather(x_user, _tmajor(neigh_u2m_0[:N_DST])).reshape(FANOUT, N_DST, D)

    w_m2u = _conv_weights(p["conv1"]["m2u"])
    w_u2m = _conv_weights(p["conv1"]["u2m"])
    lng, lnb = p["ln_g"], p["ln_b"]
    hdst1 = jnp.pad(x_user[:N_MID], ((0, n_mid_pad - N_MID), (0, 0)))
    n_split = 5
    rows = n_mid_pad // n_split
    g1_parts = [
        _sc_gather(x_message, idx1[k * rows:(k + 1) * rows].T.reshape(-1))
        .reshape(FANOUT, rows, D)
        for k in range(n_split)
    ]
    usr1_parts = [
        _conv_call(g1_parts[k], hdst1[k * rows:(k + 1) * rows],
                   w_m2u, lng, lnb, bn1)
        for k in range(n_split)
    ]
    usr1 = jnp.concatenate(usr1_parts, axis=0)
    msg1 = _conv_call(g2, x_message[:N_DST], w_u2m, lng, lnb, bn1)

    g3 = _sc_gather(usr1, _tmajor(neigh_u2m_1)).reshape(FANOUT, N_DST, D)
    w2l = _conv_weights(p["conv2"]["u2m"])
    bn_scale = 1.0 / jnp.sqrt(jnp.float32(1.0 + 1e-5))
    head = (p["bn_g"] * bn_scale, p["bn_b"], p["W1"].T, p["b1"], p["W2"][0])
    logits = _conv_call(g3, msg1, w2l, lng, lnb, bn1, head=head)
    return logits + p["b2"]


# speedup vs baseline: 2.1596x; 1.0065x over previous
import functools

import jax
import jax.numpy as jnp
from jax import lax
from jax.experimental import pallas as pl
from jax.experimental.pallas import tpu as pltpu
from jax.experimental.pallas import tpu_sc as plsc

N_SRC = 50000
N_MID = 20000
N_DST = 4096
FANOUT = 16
D = 128
H = 128

_NC = 2
_NS = 16
_NW = _NC * _NS
_CHUNK = 128


def _sc_gather(table, idx_flat):
    B = idx_flat.shape[0]
    assert B % (_NW * _CHUNK) == 0, B
    n_chunks = B // (_NW * _CHUNK)
    d = table.shape[1]
    idx2d = idx_flat.reshape(B // _CHUNK, _CHUNK)

    mesh = plsc.VectorSubcoreMesh(core_axis_name="c", subcore_axis_name="s")

    _NBUF = 4
    assert n_chunks % _NBUF == 0, n_chunks

    def body(table_hbm, idx_hbm, out_hbm, idx_v, *rest):
        bufs = rest[:_NBUF]
        gsems = rest[_NBUF:2 * _NBUF]
        wsems = rest[2 * _NBUF:3 * _NBUF]
        wid = lax.axis_index("s") * _NC + lax.axis_index("c")
        pltpu.sync_copy(idx_hbm.at[pl.ds(wid * n_chunks, n_chunks)], idx_v)

        def fire(j, p):
            pltpu.make_async_copy(table_hbm.at[idx_v.at[j]], bufs[p],
                                  gsems[p]).start()

        def out_slice(j):
            return out_hbm.at[pl.ds((wid * n_chunks + j) * _CHUNK, _CHUNK)]

        for p in range(_NBUF):
            fire(p, p)

        def step(i, carry):
            for p in range(_NBUF):
                j = _NBUF * i + p
                pltpu.make_async_copy(table_hbm.at[idx_v.at[0]], bufs[p],
                                      gsems[p]).wait()
                wb = pltpu.make_async_copy(bufs[p], out_slice(j), wsems[p])
                wb.start()
                wb.wait()

                @pl.when(j + _NBUF < n_chunks)
                def _():
                    fire(j + _NBUF, p)
            return carry

        lax.fori_loop(0, n_chunks // _NBUF, step, 0)

    gather = pl.kernel(
        body,
        out_type=jax.ShapeDtypeStruct((B, d), table.dtype),
        mesh=mesh,
        scratch_types=(
            [pltpu.VMEM((n_chunks, _CHUNK), jnp.int32)]
            + [pltpu.VMEM((_CHUNK, d), table.dtype) for _ in range(_NBUF)]
            + [pltpu.SemaphoreType.DMA for _ in range(2 * _NBUF)]
        ),
    )
    return gather(table, idx2d)


def _gelu(x):
    return 0.5 * x * (1.0 + lax.erf(x * (2.0 ** -0.5)))


def _sigmoid(x):
    return 0.5 * (1.0 + jnp.tanh(0.5 * x))


def _lstm_block(g_ref, hdst_ref, wcat_ref, bg_ref, wdn_ref, bout_ref):
    bn = g_ref.shape[1]
    h = jnp.zeros((bn, H), jnp.float32)
    c = jnp.zeros((bn, H), jnp.float32)
    for t in range(FANOUT):
        xh = jnp.concatenate([g_ref[t].astype(jnp.bfloat16),
                              h.astype(jnp.bfloat16)], axis=1)
        gates = jnp.dot(xh, wcat_ref[...], preferred_element_type=jnp.float32)
        gates = gates + bg_ref[...]
        i = _sigmoid(gates[:, 0:H])
        f = _sigmoid(gates[:, H:2 * H])
        g = jnp.tanh(gates[:, 2 * H:3 * H])
        o = _sigmoid(gates[:, 3 * H:4 * H])
        c = f * c + i * g
        h = o * jnp.tanh(c)
    dh = jnp.concatenate([hdst_ref[...], h], axis=1)
    out = jnp.dot(dh, wdn_ref[...], preferred_element_type=jnp.float32)
    return _gelu(out + bout_ref[...])


def _layernorm(x, g, b):
    m = jnp.mean(x, axis=-1, keepdims=True)
    xc = x - m
    v = jnp.mean(xc * xc, axis=-1, keepdims=True)
    return xc * lax.rsqrt(v + 1e-5) * g + b


def _conv_body(g_ref, hdst_ref, wcat_ref, bg_ref, wdn_ref, bout_ref,
               lng_ref, lnb_ref, out_ref):
    out = _lstm_block(g_ref, hdst_ref, wcat_ref, bg_ref, wdn_ref, bout_ref)
    y = _layernorm(out, lng_ref[...], lnb_ref[...])
    out_ref[...] = y.astype(out_ref.dtype)


def _conv_head_body(g_ref, hdst_ref, wcat_ref, bg_ref, wdn_ref, bout_ref,
                    lng_ref, lnb_ref, sg_ref, sb_ref, w1t_ref, b1_ref,
                    w2_ref, out_ref):
    out = _lstm_block(g_ref, hdst_ref, wcat_ref, bg_ref, wdn_ref, bout_ref)
    y = _layernorm(out, lng_ref[...], lnb_ref[...])
    hb = y * sg_ref[...] + sb_ref[...]
    h1 = _gelu(jnp.dot(hb, w1t_ref[...], preferred_element_type=jnp.float32)
               + b1_ref[...])
    out_ref[...] = jnp.sum(h1 * w2_ref[...], axis=1, keepdims=True)


def _row(v):
    return v.reshape(1, -1)


def _conv_weights(p):
    wcat = jnp.concatenate([p["W_ih"].T, p["W_hh"].T], axis=0)
    bg = _row(p["b_ih"] + p["b_hh"])
    wdn = jnp.concatenate([p["W_self"].T, p["W_neigh"].T], axis=0)
    bout = _row(p["b_self"] + p["b_neigh"])
    return wcat.astype(jnp.bfloat16), bg, wdn, bout


def _full(shape):
    return pl.BlockSpec(shape, lambda i: (0,) * len(shape))


def _conv_call(g, hdst, wts, lng, lnb, bn_rows, head=None, out_dtype=jnp.float32):
    t, n, d = g.shape
    grid = n // bn_rows
    in_specs = [
        pl.BlockSpec((t, bn_rows, d), lambda i: (0, i, 0)),
        pl.BlockSpec((bn_rows, d), lambda i: (i, 0)),
        _full((2 * D, 4 * H)),
        _full((1, 4 * H)),
        _full((2 * H, H)),
        _full((1, H)),
        _full((1, H)),
        _full((1, H)),
    ]
    args = [g, hdst, *wts, _row(lng), _row(lnb)]
    if head is None:
        body = _conv_body
        out_shape = jax.ShapeDtypeStruct((n, H), out_dtype)
        out_spec = pl.BlockSpec((bn_rows, H), lambda i: (i, 0))
    else:
        sg, sb, w1t, b1, w2 = head
        body = _conv_head_body
        in_specs += [_full((1, H)), _full((1, H)), _full((H, H)),
                     _full((1, H)), _full((1, H))]
        args += [_row(sg), _row(sb), w1t, _row(b1), _row(w2)]
        out_shape = jax.ShapeDtypeStruct((n, 1), jnp.float32)
        out_spec = pl.BlockSpec((bn_rows, 1), lambda i: (i, 0))
    return pl.pallas_call(
        body,
        grid=(grid,),
        in_specs=in_specs,
        out_specs=out_spec,
        out_shape=out_shape,
        compiler_params=pltpu.CompilerParams(
            dimension_semantics=("arbitrary",),
        ),
    )(*args)


def _tmajor(idx):
    return idx.astype(jnp.int32).T.reshape(-1)


def kernel(x_user, x_message, params, neigh_u2m_0, neigh_m2u_0,
           neigh_u2m_1, neigh_m2u_1):
    p = params
    bn1 = 512
    n_mid_pad = 20480

    n_pad_rows = n_mid_pad - N_MID
    pad_idx = (jnp.arange(n_pad_rows * FANOUT, dtype=jnp.int32) * 65)[:, None]
    pad_idx = pad_idx.reshape(n_pad_rows, FANOUT) % N_SRC
    idx1 = jnp.concatenate([neigh_m2u_0.astype(jnp.int32), pad_idx], axis=0)
    g2 = _sc_gather(x_user, _tmajor(neigh_u2m_0[:N_DST])).reshape(FANOUT, N_DST, D)

    w_m2u = _conv_weights(p["conv1"]["m2u"])
    w_u2m = _conv_weights(p["conv1"]["u2m"])
    lng, lnb = p["ln_g"], p["ln_b"]
    hdst1 = jnp.pad(x_user[:N_MID], ((0, n_mid_pad - N_MID), (0, 0)))
    splits = (2048, 2048, 4096, 4096, 4096, 4096)
    offs = [0]
    for s in splits:
        offs.append(offs[-1] + s)
    g1_parts = [
        _sc_gather(x_message, idx1[o:o + s].T.reshape(-1)).reshape(FANOUT, s, D)
        for o, s in zip(offs, splits)
    ]
    usr1_parts = []
    msg1 = None
    for k, (o, s) in enumerate(zip(offs, splits)):
        usr1_parts.append(
            _conv_call(g1_parts[k], hdst1[o:o + s], w_m2u, lng, lnb,
                       min(bn1, s)))
        if k == 2:
            msg1 = _conv_call(g2, x_message[:N_DST], w_u2m, lng, lnb, bn1)
    usr1 = jnp.concatenate(usr1_parts, axis=0)

    g3 = _sc_gather(usr1, _tmajor(neigh_u2m_1)).reshape(FANOUT, N_DST, D)
    w2l = _conv_weights(p["conv2"]["u2m"])
    bn_scale = 1.0 / jnp.sqrt(jnp.float32(1.0 + 1e-5))
    head = (p["bn_g"] * bn_scale, p["bn_b"], p["W1"].T, p["b1"], p["W2"][0])
    logits = _conv_call(g3, msg1, w2l, lng, lnb, bn1, head=head)
    return logits + p["b2"]


# speedup vs baseline: 2.1754x; 1.0073x over previous
import functools

import jax
import jax.numpy as jnp
from jax import lax
from jax.experimental import pallas as pl
from jax.experimental.pallas import tpu as pltpu
from jax.experimental.pallas import tpu_sc as plsc

N_SRC = 50000
N_MID = 20000
N_DST = 4096
FANOUT = 16
D = 128
H = 128

_NC = 2
_NS = 16
_NW = _NC * _NS
_CHUNK = 128


def _sc_gather(table, idx_flat):
    B = idx_flat.shape[0]
    assert B % (_NW * _CHUNK) == 0, B
    n_chunks = B // (_NW * _CHUNK)
    d = table.shape[1]
    idx2d = idx_flat.reshape(B // _CHUNK, _CHUNK)

    mesh = plsc.VectorSubcoreMesh(core_axis_name="c", subcore_axis_name="s")

    _NBUF = 4
    assert n_chunks % _NBUF == 0, n_chunks

    def body(table_hbm, idx_hbm, out_hbm, idx_v, *rest):
        bufs = rest[:_NBUF]
        gsems = rest[_NBUF:2 * _NBUF]
        wsems = rest[2 * _NBUF:3 * _NBUF]
        wid = lax.axis_index("s") * _NC + lax.axis_index("c")
        pltpu.sync_copy(idx_hbm.at[pl.ds(wid * n_chunks, n_chunks)], idx_v)

        def fire(j, p):
            pltpu.make_async_copy(table_hbm.at[idx_v.at[j]], bufs[p],
                                  gsems[p]).start()

        def out_slice(j):
            return out_hbm.at[pl.ds((wid * n_chunks + j) * _CHUNK, _CHUNK)]

        for p in range(_NBUF):
            fire(p, p)

        def step(i, carry):
            for p in range(_NBUF):
                j = _NBUF * i + p
                pltpu.make_async_copy(table_hbm.at[idx_v.at[0]], bufs[p],
                                      gsems[p]).wait()
                wb = pltpu.make_async_copy(bufs[p], out_slice(j), wsems[p])
                wb.start()
                wb.wait()

                @pl.when(j + _NBUF < n_chunks)
                def _():
                    fire(j + _NBUF, p)
            return carry

        lax.fori_loop(0, n_chunks // _NBUF, step, 0)

    gather = pl.kernel(
        body,
        out_type=jax.ShapeDtypeStruct((B, d), table.dtype),
        mesh=mesh,
        scratch_types=(
            [pltpu.VMEM((n_chunks, _CHUNK), jnp.int32)]
            + [pltpu.VMEM((_CHUNK, d), table.dtype) for _ in range(_NBUF)]
            + [pltpu.SemaphoreType.DMA for _ in range(2 * _NBUF)]
        ),
    )
    return gather(table, idx2d)


def _gelu(x):
    return 0.5 * x * (1.0 + lax.erf(x * (2.0 ** -0.5)))


def _sigmoid(x):
    return 0.5 * (1.0 + jnp.tanh(0.5 * x))


def _lstm_block(g_ref, hdst_ref, wcat_ref, bg_ref, wdn_ref, bout_ref):
    bn = g_ref.shape[1]
    h = jnp.zeros((bn, H), jnp.float32)
    c = jnp.zeros((bn, H), jnp.float32)
    for t in range(FANOUT):
        xh = jnp.concatenate([g_ref[t].astype(jnp.bfloat16),
                              h.astype(jnp.bfloat16)], axis=1)
        gates = jnp.dot(xh, wcat_ref[...], preferred_element_type=jnp.float32)
        gates = gates + bg_ref[...]
        i = _sigmoid(gates[:, 0:H])
        f = _sigmoid(gates[:, H:2 * H])
        g = jnp.tanh(gates[:, 2 * H:3 * H])
        o = _sigmoid(gates[:, 3 * H:4 * H])
        c = f * c + i * g
        h = o * jnp.tanh(c)
    dh = jnp.concatenate([hdst_ref[...], h], axis=1)
    out = jnp.dot(dh, wdn_ref[...], preferred_element_type=jnp.float32)
    return _gelu(out + bout_ref[...])


def _layernorm(x, g, b):
    m = jnp.mean(x, axis=-1, keepdims=True)
    xc = x - m
    v = jnp.mean(xc * xc, axis=-1, keepdims=True)
    return xc * lax.rsqrt(v + 1e-5) * g + b


def _conv_body(g_ref, hdst_ref, wcat_ref, bg_ref, wdn_ref, bout_ref,
               lng_ref, lnb_ref, out_ref):
    out = _lstm_block(g_ref, hdst_ref, wcat_ref, bg_ref, wdn_ref, bout_ref)
    y = _layernorm(out, lng_ref[...], lnb_ref[...])
    out_ref[...] = y.astype(out_ref.dtype)


def _conv_head_body(g_ref, hdst_ref, wcat_ref, bg_ref, wdn_ref, bout_ref,
                    lng_ref, lnb_ref, sg_ref, sb_ref, w1t_ref, b1_ref,
                    w2_ref, out_ref):
    out = _lstm_block(g_ref, hdst_ref, wcat_ref, bg_ref, wdn_ref, bout_ref)
    y = _layernorm(out, lng_ref[...], lnb_ref[...])
    hb = y * sg_ref[...] + sb_ref[...]
    h1 = _gelu(jnp.dot(hb, w1t_ref[...], preferred_element_type=jnp.float32)
               + b1_ref[...])
    out_ref[...] = jnp.sum(h1 * w2_ref[...], axis=1, keepdims=True)


def _row(v):
    return v.reshape(1, -1)


def _conv_weights(p):
    wcat = jnp.concatenate([p["W_ih"].T, p["W_hh"].T], axis=0)
    bg = _row(p["b_ih"] + p["b_hh"])
    wdn = jnp.concatenate([p["W_self"].T, p["W_neigh"].T], axis=0)
    bout = _row(p["b_self"] + p["b_neigh"])
    return wcat.astype(jnp.bfloat16), bg, wdn, bout


def _full(shape):
    return pl.BlockSpec(shape, lambda i: (0,) * len(shape))


def _conv_call(g, hdst, wts, lng, lnb, bn_rows, head=None, out_dtype=jnp.float32):
    t, n, d = g.shape
    grid = n // bn_rows
    in_specs = [
        pl.BlockSpec((t, bn_rows, d), lambda i: (0, i, 0)),
        pl.BlockSpec((bn_rows, d), lambda i: (i, 0)),
        _full((2 * D, 4 * H)),
        _full((1, 4 * H)),
        _full((2 * H, H)),
        _full((1, H)),
        _full((1, H)),
        _full((1, H)),
    ]
    args = [g, hdst, *wts, _row(lng), _row(lnb)]
    if head is None:
        body = _conv_body
        out_shape = jax.ShapeDtypeStruct((n, H), out_dtype)
        out_spec = pl.BlockSpec((bn_rows, H), lambda i: (i, 0))
    else:
        sg, sb, w1t, b1, w2 = head
        body = _conv_head_body
        in_specs += [_full((1, H)), _full((1, H)), _full((H, H)),
                     _full((1, H)), _full((1, H))]
        args += [_row(sg), _row(sb), w1t, _row(b1), _row(w2)]
        out_shape = jax.ShapeDtypeStruct((n, 1), jnp.float32)
        out_spec = pl.BlockSpec((bn_rows, 1), lambda i: (i, 0))
    return pl.pallas_call(
        body,
        grid=(grid,),
        in_specs=in_specs,
        out_specs=out_spec,
        out_shape=out_shape,
        compiler_params=pltpu.CompilerParams(
            dimension_semantics=("arbitrary",),
        ),
    )(*args)


def _tmajor(idx):
    return idx.astype(jnp.int32).T.reshape(-1)


def kernel(x_user, x_message, params, neigh_u2m_0, neigh_m2u_0,
           neigh_u2m_1, neigh_m2u_1):
    p = params
    bn1 = 1024
    n_mid_pad = 20480

    n_pad_rows = n_mid_pad - N_MID
    pad_idx = (jnp.arange(n_pad_rows * FANOUT, dtype=jnp.int32) * 65)[:, None]
    pad_idx = pad_idx.reshape(n_pad_rows, FANOUT) % N_SRC
    idx1 = jnp.concatenate([neigh_m2u_0.astype(jnp.int32), pad_idx], axis=0)
    g2 = _sc_gather(x_user, _tmajor(neigh_u2m_0[:N_DST])).reshape(FANOUT, N_DST, D)

    w_m2u = _conv_weights(p["conv1"]["m2u"])
    w_u2m = _conv_weights(p["conv1"]["u2m"])
    lng, lnb = p["ln_g"], p["ln_b"]
    hdst1 = jnp.pad(x_user[:N_MID], ((0, n_mid_pad - N_MID), (0, 0)))
    splits = (2048, 2048, 4096, 4096, 4096, 4096)
    offs = [0]
    for s in splits:
        offs.append(offs[-1] + s)
    g1_parts = [
        _sc_gather(x_message, idx1[o:o + s].T.reshape(-1)).reshape(FANOUT, s, D)
        for o, s in zip(offs, splits)
    ]
    usr1_parts = []
    msg1 = None
    for k, (o, s) in enumerate(zip(offs, splits)):
        usr1_parts.append(
            _conv_call(g1_parts[k], hdst1[o:o + s], w_m2u, lng, lnb,
                       min(bn1, s)))
        if k == 2:
            msg1 = _conv_call(g2, x_message[:N_DST], w_u2m, lng, lnb, bn1)
    usr1 = jnp.concatenate(usr1_parts, axis=0)

    g3 = _sc_gather(usr1, _tmajor(neigh_u2m_1)).reshape(FANOUT, N_DST, D)
    w2l = _conv_weights(p["conv2"]["u2m"])
    bn_scale = 1.0 / jnp.sqrt(jnp.float32(1.0 + 1e-5))
    head = (p["bn_g"] * bn_scale, p["bn_b"], p["W1"].T, p["b1"], p["W2"][0])
    logits = _conv_call(g3, msg1, w2l, lng, lnb, bn1, head=head)
    return logits + p["b2"]
